# Initial kernel scaffold; baseline (speedup 1.0000x reference)
import jax
import jax.numpy as jnp
from jax.experimental import pallas as pl


def kernel(f, f_info_bank, idx, contrast_idx, W1, b1, W2, b2, memory_v1, memory_v2):
    raise NotImplementedError("write your pallas kernel here")


# speedup vs baseline: 6.2776x; 6.2776x over previous
import functools

import jax
import jax.numpy as jnp
from jax import lax
from jax.experimental import pallas as pl
from jax.experimental.pallas import tpu as pltpu
from jax.experimental.pallas import tpu_sc as plsc

_EPS = 1e-07
_N_DATA = 100000
_FEAT = 2048
_CDIM = 128
_K = 512
_T = 0.07
_B = 1024

_NC = 2
_NS = 16
_NW = _NC * _NS
_NB = _B // _NW
_CH = 64
_QN = _K // _CH
_STEPS = _NB * _QN


def _embed_body(f_ref, fb_ref, w1_ref, b1_ref, w2_ref, b2_ref, fe_ref, fbe_ref):
    inv_t = 1.0 / _T

    def one(x_ref, w_ref, b_ref, o_ref):
        x = x_ref[...]
        y = lax.dot_general(x, w_ref[...], (((1,), (1,)), ((), ())),
                            preferred_element_type=jnp.float32)
        y = y + b_ref[...]
        nrm = jnp.sqrt(jnp.sum(y * y, axis=1, keepdims=True))
        o_ref[...] = y * (inv_t / nrm)

    one(f_ref, w1_ref, b1_ref, fe_ref)
    one(fb_ref, w2_ref, b2_ref, fbe_ref)


def _embed(f, fb, w1, b1, w2, b2):
    blk = 256
    grid = _B // blk
    return pl.pallas_call(
        _embed_body,
        grid=(grid,),
        in_specs=[
            pl.BlockSpec((blk, _FEAT), lambda i: (i, 0)),
            pl.BlockSpec((blk, _FEAT), lambda i: (i, 0)),
            pl.BlockSpec((_CDIM, _FEAT), lambda i: (0, 0)),
            pl.BlockSpec((1, _CDIM), lambda i: (0, 0)),
            pl.BlockSpec((_CDIM, _FEAT), lambda i: (0, 0)),
            pl.BlockSpec((1, _CDIM), lambda i: (0, 0)),
        ],
        out_specs=[
            pl.BlockSpec((blk, _CDIM), lambda i: (i, 0)),
            pl.BlockSpec((blk, _CDIM), lambda i: (i, 0)),
        ],
        out_shape=[
            jax.ShapeDtypeStruct((_B, _CDIM), jnp.float32),
            jax.ShapeDtypeStruct((_B, _CDIM), jnp.float32),
        ],
    )(f, fb, w1, b1, w2, b2)


def _scr_reduce(scr_ref):
    lanes = lax.iota(jnp.int32, 16)
    tot = plsc.load_gather(scr_ref, [lanes, jnp.zeros((16,), jnp.int32)])
    for c in range(1, 16):
        tot = tot + plsc.load_gather(scr_ref, [lanes, jnp.full((16,), c, jnp.int32)])
    return tot


def _rows16_dot(buf_ref, g, fvecs, scr_ref):
    for u in range(16):
        r = g * 16 + u
        acc = buf_ref[r, pl.ds(0, 16)] * fvecs[0]
        for j in range(1, 8):
            acc = acc + buf_ref[r, pl.ds(16 * j, 16)] * fvecs[j]
        scr_ref[u] = acc
    return _scr_reduce(scr_ref)


def _rows16_dot_perrow(buf_ref, g, f_ref, scr_ref):
    for u in range(16):
        r = g * 16 + u
        acc = buf_ref[r, pl.ds(0, 16)] * f_ref[r, pl.ds(0, 16)]
        for j in range(1, 8):
            acc = acc + buf_ref[r, pl.ds(16 * j, 16)] * f_ref[r, pl.ds(16 * j, 16)]
        scr_ref[u] = acc
    return _scr_reduce(scr_ref)


def _sc_body(mem1, mem2, fb1, fb2, idxp, idxn,
             pos1_o, pos2_o, neg1_o, neg2_o,
             f1_v, f2_v, idxp_v, idxn_v, prow1_v, prow2_v,
             bufa1, bufb1, bufa2, bufb2,
             neg1_v, neg2_v, pos1_v, pos2_v, scr_v,
             sema1, semb1, sema2, semb2):
    wid = lax.axis_index("s") * _NC + lax.axis_index("c")
    base = wid * _NB

    pltpu.sync_copy(fb1.at[pl.ds(base, _NB)], f1_v)
    pltpu.sync_copy(fb2.at[pl.ds(base, _NB)], f2_v)
    pltpu.sync_copy(idxp.at[pl.ds(base, _NB)], idxp_v)
    pltpu.sync_copy(idxn.at[wid], idxn_v)

    def issue(s, b1buf, b2buf, s1, s2):
        b = s // _QN
        q = s - b * _QN
        row = idxn_v.at[b, q]
        pltpu.async_copy(mem1.at[row], b1buf, s1)
        pltpu.async_copy(mem2.at[row], b2buf, s2)

    def drain(b1buf, b2buf, s1, s2):
        pltpu.make_async_copy(mem1.at[pl.ds(0, _CH)], b1buf, s1).wait()
        pltpu.make_async_copy(mem2.at[pl.ds(0, _CH)], b2buf, s2).wait()

    pltpu.async_copy(mem1.at[idxp_v], prow1_v, semb1)
    pltpu.async_copy(mem2.at[idxp_v], prow2_v, semb2)
    issue(0, bufa1, bufa2, sema1, sema2)
    pltpu.make_async_copy(mem1.at[pl.ds(0, _NB)], prow1_v, semb1).wait()
    pltpu.make_async_copy(mem2.at[pl.ds(0, _NB)], prow2_v, semb2).wait()
    for g in range(_NB // 16):
        pos1_v[pl.ds(16 * g, 16)] = _rows16_dot_perrow(prow1_v, g, f1_v, scr_v)
        pos2_v[pl.ds(16 * g, 16)] = _rows16_dot_perrow(prow2_v, g, f2_v, scr_v)
    issue(1, bufb1, bufb2, semb1, semb2)

    def compute(s, b1buf, b2buf):
        b = s // _QN
        q = s - b * _QN
        f1vecs = [f1_v[b, pl.ds(16 * j, 16)] for j in range(8)]
        f2vecs = [f2_v[b, pl.ds(16 * j, 16)] for j in range(8)]

        def group(g, _):
            neg1_v[b, pl.ds(q * _CH + 16 * g, 16)] = _rows16_dot(b1buf, g, f1vecs, scr_v)
            neg2_v[b, pl.ds(q * _CH + 16 * g, 16)] = _rows16_dot(b2buf, g, f2vecs, scr_v)
            return 0

        lax.fori_loop(0, _CH // 16, group, 0)

    def two_steps(t, _):
        s0 = 2 * t
        s1 = s0 + 1
        drain(bufa1, bufa2, sema1, sema2)
        compute(s0, bufa1, bufa2)

        @pl.when(s0 + 2 < _STEPS)
        def _():
            issue(s0 + 2, bufa1, bufa2, sema1, sema2)

        drain(bufb1, bufb2, semb1, semb2)
        compute(s1, bufb1, bufb2)

        @pl.when(s1 + 2 < _STEPS)
        def _():
            issue(s1 + 2, bufb1, bufb2, semb1, semb2)

        return 0

    lax.fori_loop(0, _STEPS // 2, two_steps, 0)

    pltpu.sync_copy(pos1_v, pos1_o.at[pl.ds(base, _NB)])
    pltpu.sync_copy(pos2_v, pos2_o.at[pl.ds(base, _NB)])
    pltpu.sync_copy(neg1_v, neg1_o.at[pl.ds(base, _NB)])
    pltpu.sync_copy(neg2_v, neg2_o.at[pl.ds(base, _NB)])


@functools.cache
def _sc_scores_call():
  return functools.partial(
    pl.kernel,
    mesh=plsc.VectorSubcoreMesh(core_axis_name="c", subcore_axis_name="s",
                                num_cores=_NC, num_subcores=_NS),
    compiler_params=pltpu.CompilerParams(needs_layout_passes=False),
    out_type=(
        jax.ShapeDtypeStruct((_B,), jnp.float32),
        jax.ShapeDtypeStruct((_B,), jnp.float32),
        jax.ShapeDtypeStruct((_B, _K), jnp.float32),
        jax.ShapeDtypeStruct((_B, _K), jnp.float32),
    ),
    scratch_types=[
        pltpu.VMEM((_NB, _CDIM), jnp.float32),
        pltpu.VMEM((_NB, _CDIM), jnp.float32),
        pltpu.VMEM((_NB,), jnp.int32),
        pltpu.VMEM((_NB, _QN, _CH), jnp.int32),
        pltpu.VMEM((_NB, _CDIM), jnp.float32),
        pltpu.VMEM((_NB, _CDIM), jnp.float32),
        pltpu.VMEM((_CH, _CDIM), jnp.float32),
        pltpu.VMEM((_CH, _CDIM), jnp.float32),
        pltpu.VMEM((_CH, _CDIM), jnp.float32),
        pltpu.VMEM((_CH, _CDIM), jnp.float32),
        pltpu.VMEM((_NB, _K), jnp.float32),
        pltpu.VMEM((_NB, _K), jnp.float32),
        pltpu.VMEM((_NB,), jnp.float32),
        pltpu.VMEM((_NB,), jnp.float32),
        pltpu.VMEM((16, 16), jnp.float32),
        pltpu.SemaphoreType.DMA,
        pltpu.SemaphoreType.DMA,
        pltpu.SemaphoreType.DMA,
        pltpu.SemaphoreType.DMA,
    ],
  )(_sc_body)


def _loss_body(p1_ref, n1_ref, p2_ref, n2_ref, o_ref):
    cn = float(_K) / float(_N_DATA)

    def view(p_ref, n_ref):
        ps = p_ref[...]
        ns = n_ref[...]
        ep = jnp.exp(ps)
        en = jnp.exp(ns)
        s_tot = jnp.sum(ep) + jnp.sum(en)
        z = s_tot / float(_B * (_K + 1)) * float(_N_DATA)
        t_all = (jnp.sum(jnp.log(ep / z + (cn + _EPS)))
                 + jnp.sum(jnp.log(en / z + (cn + _EPS))))
        sig = (jnp.sum(ps) - float(_B) * jnp.log(z)
               + float(_B * _K) * jnp.log(cn) - t_all)
        return -sig / float(_B)

    o_ref[...] = jnp.full((1, 1), view(p1_ref, n1_ref) + view(p2_ref, n2_ref),
                          jnp.float32)


def _loss(p1, n1, p2, n2):
    return pl.pallas_call(
        _loss_body,
        out_shape=jax.ShapeDtypeStruct((1, 1), jnp.float32),
    )(p1, n1, p2, n2)


def kernel(f, f_info_bank, idx, contrast_idx, W1, b1, W2, b2, memory_v1, memory_v2):
    fe, fbe = _embed(f, f_info_bank, W1, b1.reshape(1, _CDIM), W2, b2.reshape(1, _CDIM))
    idx_neg = contrast_idx[:, 1:].reshape(_NW, _NB, _QN, _CH)
    pos1, pos2, neg1, neg2 = _sc_scores_call()(memory_v1, memory_v2, fbe, fe,
                                               idx, idx_neg)
    out = _loss(pos1.reshape(8, _CDIM), neg1, pos2.reshape(8, _CDIM), neg2)
    return out[0]
